# baseline (speedup 1.0000x reference)
import functools

import jax
import jax.numpy as jnp
from jax import lax
from jax.experimental import pallas as pl
from jax.experimental.pallas import tpu as pltpu
from jax.experimental.pallas import tpu_sc as plsc

NC = 2
NS = 16
NW = NC * NS
CH = 128

_MESH = dict(core_axis_name="c", subcore_axis_name="s")
_CP = pltpu.CompilerParams(use_tc_tiling_on_sc=False)


def _deg_call(dst_all, ones, zeros, npad, nchunk):
    rps = npad // NS
    mesh = plsc.VectorSubcoreMesh(num_cores=NC, **_MESH)

    @functools.partial(
        pl.kernel,
        mesh=mesh,
        compiler_params=_CP,
        out_type=jax.ShapeDtypeStruct((NC * npad, 16), jnp.float32),
        scratch_types=[
            pltpu.VMEM((nchunk, CH), jnp.int32),
            pltpu.VMEM((CH, 16), jnp.float32),
            pltpu.VMEM_SHARED((npad, 16), jnp.float32),
        ],
    )
    def degk(dst_hbm, ones_hbm, z_hbm, out_hbm, dst_v, ones_v, acc_sh):
        c = lax.axis_index("c")
        s = lax.axis_index("s")
        wid = s * NC + c
        pltpu.sync_copy(z_hbm.at[pl.ds(s * rps, rps)],
                        acc_sh.at[pl.ds(s * rps, rps)])
        pltpu.sync_copy(ones_hbm, ones_v)
        pltpu.sync_copy(dst_hbm.at[wid], dst_v)
        plsc.subcore_barrier()

        @pl.loop(0, nchunk)
        def _(j):
            pltpu.sync_copy(ones_v, acc_sh.at[dst_v.at[j]], add=True)

        plsc.subcore_barrier()
        pltpu.sync_copy(acc_sh.at[pl.ds(s * rps, rps)],
                        out_hbm.at[pl.ds(c * npad + s * rps, rps)])

    return degk(dst_all, ones, zeros)


def _prop_call(g, src_all, dst_all, zeros, npad, nchunk, d):
    rps = npad // NS
    mesh = plsc.VectorSubcoreMesh(num_cores=1, **_MESH)

    @functools.partial(
        pl.kernel,
        mesh=mesh,
        compiler_params=_CP,
        out_type=jax.ShapeDtypeStruct((npad, d), jnp.float32),
        scratch_types=[
            pltpu.VMEM((nchunk, CH), jnp.int32),
            pltpu.VMEM((nchunk, CH), jnp.int32),
            pltpu.VMEM((CH, d), jnp.float32),
            pltpu.VMEM((CH, d), jnp.float32),
            pltpu.VMEM_SHARED((npad, d), jnp.float32),
            pltpu.VMEM_SHARED((npad, d), jnp.float32),
            pltpu.SemaphoreType.DMA,
            pltpu.SemaphoreType.DMA,
        ],
    )
    def prop(g_hbm, src_hbm, dst_hbm, z_hbm, out_hbm,
             src_v, dst_v, buf0_v, buf1_v, table_sh, acc_sh, sem0, sem1):
        wid = lax.axis_index("s")
        pltpu.sync_copy(g_hbm.at[pl.ds(wid * rps, rps)],
                        table_sh.at[pl.ds(wid * rps, rps)])
        pltpu.sync_copy(z_hbm.at[pl.ds(wid * rps, rps)],
                        acc_sh.at[pl.ds(wid * rps, rps)])
        pltpu.sync_copy(src_hbm.at[wid], src_v)
        pltpu.sync_copy(dst_hbm.at[wid], dst_v)
        plsc.subcore_barrier()

        @pl.loop(0, nchunk, step=2)
        def _(j):
            cp0 = pltpu.async_copy(table_sh.at[src_v.at[j]], buf0_v, sem0)
            cp1 = pltpu.async_copy(table_sh.at[src_v.at[j + 1]], buf1_v, sem1)
            cp0.wait()
            pltpu.sync_copy(buf0_v, acc_sh.at[dst_v.at[j]], add=True)
            cp1.wait()
            pltpu.sync_copy(buf1_v, acc_sh.at[dst_v.at[j + 1]], add=True)

        plsc.subcore_barrier()
        pltpu.sync_copy(acc_sh.at[pl.ds(wid * rps, rps)],
                        out_hbm.at[pl.ds(wid * rps, rps)])

    return prop(g, src_all, dst_all, zeros)


def _tc_matmul1(x, w):
    def body(x_ref, w_ref, o_ref):
        o_ref[...] = lax.dot_general(
            x_ref[...], w_ref[...], (((1,), (0,)), ((), ())),
            precision=lax.Precision.HIGHEST,
            preferred_element_type=jnp.float32)

    return pl.pallas_call(
        body,
        out_shape=jax.ShapeDtypeStruct((x.shape[0], w.shape[1]), jnp.float32),
    )(x, w)


def _tc_scale1(degparts, h1, npad):
    n, dh = h1.shape

    def body(dp_ref, h_ref, g_ref, dinv_ref):
        deg = sum(dp_ref[i * npad:i * npad + n, 0:1] for i in range(NC)) + 1.0
        dinv = lax.rsqrt(deg)
        dinv_ref[...] = dinv
        g_ref[...] = h_ref[...] * dinv

    return pl.pallas_call(
        body,
        out_shape=[
            jax.ShapeDtypeStruct((n, dh), jnp.float32),
            jax.ShapeDtypeStruct((n, 1), jnp.float32),
        ],
    )(degparts, h1)


def _tc_layer2(s1, g1, dinv, b1, w2, npad):
    n, dh = g1.shape

    def body(s_ref, g_ref, di_ref, b_ref, w_ref, o_ref):
        ssum = sum(s_ref[i * npad:i * npad + n, 0:dh] for i in range(NC))
        out1 = (ssum + g_ref[...]) * di_ref[...] + b_ref[...]
        h2 = lax.dot_general(
            jnp.maximum(out1, 0.0), w_ref[...], (((1,), (0,)), ((), ())),
            precision=lax.Precision.HIGHEST,
            preferred_element_type=jnp.float32)
        o_ref[...] = h2 * di_ref[...]

    return pl.pallas_call(
        body,
        out_shape=jax.ShapeDtypeStruct((n, w2.shape[1]), jnp.float32),
    )(s1, g1, dinv, b1, w2)


def _tc_final(s2, g2, dinv, b2, npad):
    n, do = g2.shape

    def body(s_ref, g_ref, di_ref, b_ref, o_ref):
        ssum = sum(s_ref[i * npad:i * npad + n, 0:do] for i in range(NC))
        o_ref[...] = (ssum + g_ref[...]) * di_ref[...] + b_ref[...]

    return pl.pallas_call(
        body,
        out_shape=jax.ShapeDtypeStruct((n, do), jnp.float32),
    )(s2, g2, dinv, b2)


def kernel(x, edge_index, W1, b1, W2, b2):
    n = x.shape[0]
    e = edge_index.shape[1]
    npad = ((n + 1 + 127) // 128) * 128
    junk = npad - n

    ew = -(-e // NW)
    ewp = -(-ew // (2 * CH)) * (2 * CH)
    pad = NW * ewp - e
    ar = jnp.arange(pad, dtype=jnp.int32)
    src_all = jnp.concatenate([edge_index[0], ar % n])
    dst_all = jnp.concatenate([edge_index[1], n + (ar % junk)])
    nchunk = ewp // CH
    src_all = src_all.reshape(NW, nchunk, CH)
    dst_all = dst_all.reshape(NW, nchunk, CH)

    ones = jnp.ones((CH, 16), jnp.float32)
    z16 = jnp.zeros((npad, 16), jnp.float32)
    z32 = jnp.zeros((npad, 32), jnp.float32)

    degparts = _deg_call(dst_all, ones, z16, npad, nchunk)
    h1 = _tc_matmul1(x, W1)
    g1, dinv = _tc_scale1(degparts, h1, npad)
    g1p = jnp.pad(g1, ((0, npad - n), (0, 32 - g1.shape[1])))
    s1a = _prop_call(g1p, src_all[:NS], dst_all[:NS], z32, npad, nchunk, 32)
    s1b = _prop_call(g1p, src_all[NS:], dst_all[NS:], z32, npad, nchunk, 32)
    s1 = jnp.concatenate([s1a, s1b], axis=0)
    g2 = _tc_layer2(s1, g1, dinv, b1.reshape(1, -1), W2, npad)
    g2p = jnp.pad(g2, ((0, npad - n), (0, 16 - g2.shape[1])))
    s2a = _prop_call(g2p, src_all[:NS], dst_all[:NS], z16, npad, nchunk, 16)
    s2b = _prop_call(g2p, src_all[NS:], dst_all[NS:], z16, npad, nchunk, 16)
    s2 = jnp.concatenate([s2a, s2b], axis=0)
    return _tc_final(s2, g2, dinv, b2.reshape(1, -1), npad)


# speedup vs baseline: 1.6767x; 1.6767x over previous
import functools

import jax
import jax.numpy as jnp
from jax import lax
from jax.experimental import pallas as pl
from jax.experimental.pallas import tpu as pltpu
from jax.experimental.pallas import tpu_sc as plsc

NC = 2
NS = 16
NW = NC * NS
CH = 128

_MESH = dict(core_axis_name="c", subcore_axis_name="s")
_CP = pltpu.CompilerParams(use_tc_tiling_on_sc=False)


def _deg_call(dst_all, ones, zeros, npad, nchunk):
    rps = npad // NS
    mesh = plsc.VectorSubcoreMesh(num_cores=NC, **_MESH)

    @functools.partial(
        pl.kernel,
        mesh=mesh,
        compiler_params=_CP,
        out_type=jax.ShapeDtypeStruct((NC * npad, 16), jnp.float32),
        scratch_types=[
            pltpu.VMEM((nchunk, CH), jnp.int32),
            pltpu.VMEM((CH, 16), jnp.float32),
            pltpu.VMEM_SHARED((npad, 16), jnp.float32),
        ],
    )
    def degk(dst_hbm, ones_hbm, z_hbm, out_hbm, dst_v, ones_v, acc_sh):
        c = lax.axis_index("c")
        s = lax.axis_index("s")
        wid = s * NC + c
        pltpu.sync_copy(z_hbm.at[pl.ds(s * rps, rps)],
                        acc_sh.at[pl.ds(s * rps, rps)])
        pltpu.sync_copy(ones_hbm, ones_v)
        pltpu.sync_copy(dst_hbm.at[wid], dst_v)
        plsc.subcore_barrier()

        @pl.loop(0, nchunk)
        def _(j):
            pltpu.sync_copy(ones_v, acc_sh.at[dst_v.at[j]], add=True)

        plsc.subcore_barrier()
        pltpu.sync_copy(acc_sh.at[pl.ds(s * rps, rps)],
                        out_hbm.at[pl.ds(c * npad + s * rps, rps)])

    return degk(dst_all, ones, zeros)


def _prop_call(g, src_all, dst_all, zeros, npad, nchunk, d, dtype):
    rps = npad // NS
    mesh = plsc.VectorSubcoreMesh(num_cores=NC, **_MESH)

    @functools.partial(
        pl.kernel,
        mesh=mesh,
        compiler_params=_CP,
        out_type=jax.ShapeDtypeStruct((NC * npad, d), dtype),
        scratch_types=[
            pltpu.VMEM((nchunk, CH), jnp.int32),
            pltpu.VMEM((nchunk, CH), jnp.int32),
            pltpu.VMEM((CH, d), dtype),
            pltpu.VMEM((CH, d), dtype),
            pltpu.VMEM_SHARED((npad, d), dtype),
            pltpu.VMEM_SHARED((npad, d), dtype),
            pltpu.SemaphoreType.DMA,
            pltpu.SemaphoreType.DMA,
        ],
    )
    def prop(g_hbm, src_hbm, dst_hbm, z_hbm, out_hbm,
             src_v, dst_v, buf0_v, buf1_v, table_sh, acc_sh, sem0, sem1):
        c = lax.axis_index("c")
        s = lax.axis_index("s")
        wid = s * NC + c
        pltpu.sync_copy(g_hbm.at[pl.ds(s * rps, rps)],
                        table_sh.at[pl.ds(s * rps, rps)])
        pltpu.sync_copy(z_hbm.at[pl.ds(s * rps, rps)],
                        acc_sh.at[pl.ds(s * rps, rps)])
        pltpu.sync_copy(src_hbm.at[wid], src_v)
        pltpu.sync_copy(dst_hbm.at[wid], dst_v)
        plsc.subcore_barrier()

        @pl.loop(0, nchunk, step=2)
        def _(j):
            cp0 = pltpu.async_copy(table_sh.at[src_v.at[j]], buf0_v, sem0)
            cp1 = pltpu.async_copy(table_sh.at[src_v.at[j + 1]], buf1_v, sem1)
            cp0.wait()
            pltpu.sync_copy(buf0_v, acc_sh.at[dst_v.at[j]], add=True)
            cp1.wait()
            pltpu.sync_copy(buf1_v, acc_sh.at[dst_v.at[j + 1]], add=True)

        plsc.subcore_barrier()
        pltpu.sync_copy(acc_sh.at[pl.ds(s * rps, rps)],
                        out_hbm.at[pl.ds(c * npad + s * rps, rps)])

    return prop(g, src_all, dst_all, zeros)


def _tc_matmul1(x, w):
    def body(x_ref, w_ref, o_ref):
        o_ref[...] = lax.dot_general(
            x_ref[...], w_ref[...], (((1,), (0,)), ((), ())),
            precision=lax.Precision.HIGHEST,
            preferred_element_type=jnp.float32)

    return pl.pallas_call(
        body,
        out_shape=jax.ShapeDtypeStruct((x.shape[0], w.shape[1]), jnp.float32),
    )(x, w)


def _tc_scale1(degparts, h1, npad):
    n, dh = h1.shape

    def body(dp_ref, h_ref, g_ref, dinv_ref):
        deg = sum(dp_ref[i * npad:i * npad + n, 0:1] for i in range(NC)) + 1.0
        dinv = lax.rsqrt(deg)
        dinv_ref[...] = dinv
        g_ref[...] = h_ref[...] * dinv

    return pl.pallas_call(
        body,
        out_shape=[
            jax.ShapeDtypeStruct((n, dh), jnp.float32),
            jax.ShapeDtypeStruct((n, 1), jnp.float32),
        ],
    )(degparts, h1)


def _tc_layer2(s1, g1, dinv, b1, w2, npad):
    n, dh = g1.shape

    def body(s_ref, g_ref, di_ref, b_ref, w_ref, o_ref):
        ssum = sum(s_ref[i * npad:i * npad + n, 0:dh].astype(jnp.float32)
                   for i in range(NC))
        out1 = (ssum + g_ref[...]) * di_ref[...] + b_ref[...]
        h2 = lax.dot_general(
            jnp.maximum(out1, 0.0), w_ref[...], (((1,), (0,)), ((), ())),
            precision=lax.Precision.HIGHEST,
            preferred_element_type=jnp.float32)
        o_ref[...] = h2 * di_ref[...]

    return pl.pallas_call(
        body,
        out_shape=jax.ShapeDtypeStruct((n, w2.shape[1]), jnp.float32),
    )(s1, g1, dinv, b1, w2)


def _tc_final(s2, g2, dinv, b2, npad):
    n, do = g2.shape

    def body(s_ref, g_ref, di_ref, b_ref, o_ref):
        ssum = sum(s_ref[i * npad:i * npad + n, 0:do] for i in range(NC))
        o_ref[...] = (ssum + g_ref[...]) * di_ref[...] + b_ref[...]

    return pl.pallas_call(
        body,
        out_shape=jax.ShapeDtypeStruct((n, do), jnp.float32),
    )(s2, g2, dinv, b2)


def kernel(x, edge_index, W1, b1, W2, b2):
    n = x.shape[0]
    e = edge_index.shape[1]
    npad = ((n + 1 + 127) // 128) * 128
    junk = npad - n

    ew = -(-e // NW)
    ewp = -(-ew // (2 * CH)) * (2 * CH)
    pad = NW * ewp - e
    ar = jnp.arange(pad, dtype=jnp.int32)
    src_all = jnp.concatenate([edge_index[0], ar % n])
    dst_all = jnp.concatenate([edge_index[1], n + (ar % junk)])
    nchunk = ewp // CH
    src_all = src_all.reshape(NW, nchunk, CH)
    dst_all = dst_all.reshape(NW, nchunk, CH)

    ones = jnp.ones((CH, 16), jnp.float32)
    z16 = jnp.zeros((npad, 16), jnp.float32)

    degparts = _deg_call(dst_all, ones, z16, npad, nchunk)
    h1 = _tc_matmul1(x, W1)
    g1, dinv = _tc_scale1(degparts, h1, npad)
    g1p = jnp.pad(g1, ((0, npad - n), (0, 32 - g1.shape[1]))).astype(jnp.bfloat16)
    zb32 = jnp.zeros((npad, 32), jnp.bfloat16)
    s1 = _prop_call(g1p, src_all, dst_all, zb32, npad, nchunk, 32,
                    jnp.bfloat16)
    g2 = _tc_layer2(s1, g1, dinv, b1.reshape(1, -1), W2, npad)
    g2p = jnp.pad(g2, ((0, npad - n), (0, 16 - g2.shape[1])))
    s2 = _prop_call(g2p, src_all, dst_all, z16, npad, nchunk, 16,
                    jnp.float32)
    return _tc_final(s2, g2, dinv, b2.reshape(1, -1), npad)


# speedup vs baseline: 1.7637x; 1.0519x over previous
import functools

import jax
import jax.numpy as jnp
from jax import lax
from jax.experimental import pallas as pl
from jax.experimental.pallas import tpu as pltpu
from jax.experimental.pallas import tpu_sc as plsc

NC = 2
NS = 16
NW = NC * NS
CH = 512

_MESH = dict(core_axis_name="c", subcore_axis_name="s")
_CP = pltpu.CompilerParams(use_tc_tiling_on_sc=False)


def _deg_call(dst_all, ones, zeros, npad, nchunk):
    rps = npad // NS
    mesh = plsc.VectorSubcoreMesh(num_cores=NC, **_MESH)

    @functools.partial(
        pl.kernel,
        mesh=mesh,
        compiler_params=_CP,
        out_type=jax.ShapeDtypeStruct((NC * npad, 16), jnp.float32),
        scratch_types=[
            pltpu.VMEM((nchunk, CH), jnp.int32),
            pltpu.VMEM((CH, 16), jnp.float32),
            pltpu.VMEM_SHARED((npad, 16), jnp.float32),
        ],
    )
    def degk(dst_hbm, ones_hbm, z_hbm, out_hbm, dst_v, ones_v, acc_sh):
        c = lax.axis_index("c")
        s = lax.axis_index("s")
        wid = s * NC + c
        pltpu.sync_copy(z_hbm.at[pl.ds(s * rps, rps)],
                        acc_sh.at[pl.ds(s * rps, rps)])
        pltpu.sync_copy(ones_hbm, ones_v)
        pltpu.sync_copy(dst_hbm.at[wid], dst_v)
        plsc.subcore_barrier()

        @pl.loop(0, nchunk)
        def _(j):
            pltpu.sync_copy(ones_v, acc_sh.at[dst_v.at[j]], add=True)

        plsc.subcore_barrier()
        pltpu.sync_copy(acc_sh.at[pl.ds(s * rps, rps)],
                        out_hbm.at[pl.ds(c * npad + s * rps, rps)])

    return degk(dst_all, ones, zeros)


def _prop_call(g, src_all, dst_all, zeros, npad, nchunk, d, dtype):
    rps = npad // NS
    mesh = plsc.VectorSubcoreMesh(num_cores=NC, **_MESH)

    @functools.partial(
        pl.kernel,
        mesh=mesh,
        compiler_params=_CP,
        out_type=jax.ShapeDtypeStruct((NC * npad, d), dtype),
        scratch_types=[
            pltpu.VMEM((nchunk, CH), jnp.int32),
            pltpu.VMEM((nchunk, CH), jnp.int32),
            pltpu.VMEM((CH, d), dtype),
            pltpu.VMEM((CH, d), dtype),
            pltpu.VMEM_SHARED((npad, d), dtype),
            pltpu.VMEM_SHARED((npad, d), dtype),
            pltpu.SemaphoreType.DMA,
            pltpu.SemaphoreType.DMA,
        ],
    )
    def prop(g_hbm, src_hbm, dst_hbm, z_hbm, out_hbm,
             src_v, dst_v, buf0_v, buf1_v, table_sh, acc_sh, sem0, sem1):
        c = lax.axis_index("c")
        s = lax.axis_index("s")
        wid = s * NC + c
        pltpu.sync_copy(g_hbm.at[pl.ds(s * rps, rps)],
                        table_sh.at[pl.ds(s * rps, rps)])
        pltpu.sync_copy(z_hbm.at[pl.ds(s * rps, rps)],
                        acc_sh.at[pl.ds(s * rps, rps)])
        pltpu.sync_copy(src_hbm.at[wid], src_v)
        pltpu.sync_copy(dst_hbm.at[wid], dst_v)
        plsc.subcore_barrier()

        @pl.loop(0, nchunk, step=2)
        def _(j):
            cp0 = pltpu.async_copy(table_sh.at[src_v.at[j]], buf0_v, sem0)
            cp1 = pltpu.async_copy(table_sh.at[src_v.at[j + 1]], buf1_v, sem1)
            cp0.wait()
            pltpu.sync_copy(buf0_v, acc_sh.at[dst_v.at[j]], add=True)
            cp1.wait()
            pltpu.sync_copy(buf1_v, acc_sh.at[dst_v.at[j + 1]], add=True)

        plsc.subcore_barrier()
        pltpu.sync_copy(acc_sh.at[pl.ds(s * rps, rps)],
                        out_hbm.at[pl.ds(c * npad + s * rps, rps)])

    return prop(g, src_all, dst_all, zeros)


def _tc_matmul1(x, w):
    def body(x_ref, w_ref, o_ref):
        o_ref[...] = lax.dot_general(
            x_ref[...], w_ref[...], (((1,), (0,)), ((), ())),
            precision=lax.Precision.HIGHEST,
            preferred_element_type=jnp.float32)

    return pl.pallas_call(
        body,
        out_shape=jax.ShapeDtypeStruct((x.shape[0], w.shape[1]), jnp.float32),
    )(x, w)


def _tc_scale1(degparts, h1, npad):
    n, dh = h1.shape

    def body(dp_ref, h_ref, g_ref, dinv_ref):
        deg = sum(dp_ref[i * npad:i * npad + n, 0:1] for i in range(NC)) + 1.0
        dinv = lax.rsqrt(deg)
        dinv_ref[...] = dinv
        g_ref[...] = h_ref[...] * dinv

    return pl.pallas_call(
        body,
        out_shape=[
            jax.ShapeDtypeStruct((n, dh), jnp.float32),
            jax.ShapeDtypeStruct((n, 1), jnp.float32),
        ],
    )(degparts, h1)


def _tc_layer2(s1, g1, dinv, b1, w2, npad):
    n, dh = g1.shape

    def body(s_ref, g_ref, di_ref, b_ref, w_ref, o_ref):
        ssum = sum(s_ref[i * npad:i * npad + n, 0:dh].astype(jnp.float32)
                   for i in range(NC))
        out1 = (ssum + g_ref[...]) * di_ref[...] + b_ref[...]
        h2 = lax.dot_general(
            jnp.maximum(out1, 0.0), w_ref[...], (((1,), (0,)), ((), ())),
            precision=lax.Precision.HIGHEST,
            preferred_element_type=jnp.float32)
        o_ref[...] = h2 * di_ref[...]

    return pl.pallas_call(
        body,
        out_shape=jax.ShapeDtypeStruct((n, w2.shape[1]), jnp.float32),
    )(s1, g1, dinv, b1, w2)


def _tc_final(s2, g2, dinv, b2, npad):
    n, do = g2.shape

    def body(s_ref, g_ref, di_ref, b_ref, o_ref):
        ssum = sum(s_ref[i * npad:i * npad + n, 0:do] for i in range(NC))
        o_ref[...] = (ssum + g_ref[...]) * di_ref[...] + b_ref[...]

    return pl.pallas_call(
        body,
        out_shape=jax.ShapeDtypeStruct((n, do), jnp.float32),
    )(s2, g2, dinv, b2)


def kernel(x, edge_index, W1, b1, W2, b2):
    n = x.shape[0]
    e = edge_index.shape[1]
    npad = ((n + 1 + 127) // 128) * 128
    junk = npad - n

    ew = -(-e // NW)
    ewp = -(-ew // (2 * CH)) * (2 * CH)
    pad = NW * ewp - e
    ar = jnp.arange(pad, dtype=jnp.int32)
    src_all = jnp.concatenate([edge_index[0], ar % n])
    dst_all = jnp.concatenate([edge_index[1], n + (ar % junk)])
    nchunk = ewp // CH
    src_all = src_all.reshape(NW, nchunk, CH)
    dst_all = dst_all.reshape(NW, nchunk, CH)

    ones = jnp.ones((CH, 16), jnp.float32)
    z16 = jnp.zeros((npad, 16), jnp.float32)

    degparts = _deg_call(dst_all, ones, z16, npad, nchunk)
    h1 = _tc_matmul1(x, W1)
    g1, dinv = _tc_scale1(degparts, h1, npad)
    g1p = jnp.pad(g1, ((0, npad - n), (0, 32 - g1.shape[1]))).astype(jnp.bfloat16)
    zb32 = jnp.zeros((npad, 32), jnp.bfloat16)
    s1 = _prop_call(g1p, src_all, dst_all, zb32, npad, nchunk, 32,
                    jnp.bfloat16)
    g2 = _tc_layer2(s1, g1, dinv, b1.reshape(1, -1), W2, npad)
    g2p = jnp.pad(g2, ((0, npad - n), (0, 16 - g2.shape[1])))
    s2 = _prop_call(g2p, src_all, dst_all, z16, npad, nchunk, 16,
                    jnp.float32)
    return _tc_final(s2, g2, dinv, b2.reshape(1, -1), npad)


# speedup vs baseline: 1.8419x; 1.0444x over previous
import functools

import jax
import jax.numpy as jnp
from jax import lax
from jax.experimental import pallas as pl
from jax.experimental.pallas import tpu as pltpu
from jax.experimental.pallas import tpu_sc as plsc

NC = 2
NS = 16
NW = NC * NS
CH = 1024

_MESH = dict(core_axis_name="c", subcore_axis_name="s")
_CP = pltpu.CompilerParams(use_tc_tiling_on_sc=False)


def _deg_call(dst_all, ones, zeros, npad, nchunk):
    rps = npad // NS
    mesh = plsc.VectorSubcoreMesh(num_cores=NC, **_MESH)

    @functools.partial(
        pl.kernel,
        mesh=mesh,
        compiler_params=_CP,
        out_type=jax.ShapeDtypeStruct((NC * npad, 16), jnp.float32),
        scratch_types=[
            pltpu.VMEM((nchunk, CH), jnp.int32),
            pltpu.VMEM((CH, 16), jnp.float32),
            pltpu.VMEM_SHARED((npad, 16), jnp.float32),
        ],
    )
    def degk(dst_hbm, ones_hbm, z_hbm, out_hbm, dst_v, ones_v, acc_sh):
        c = lax.axis_index("c")
        s = lax.axis_index("s")
        wid = s * NC + c
        pltpu.sync_copy(z_hbm.at[pl.ds(s * rps, rps)],
                        acc_sh.at[pl.ds(s * rps, rps)])
        pltpu.sync_copy(ones_hbm, ones_v)
        pltpu.sync_copy(dst_hbm.at[wid], dst_v)
        plsc.subcore_barrier()

        @pl.loop(0, nchunk)
        def _(j):
            pltpu.sync_copy(ones_v, acc_sh.at[dst_v.at[j]], add=True)

        plsc.subcore_barrier()
        pltpu.sync_copy(acc_sh.at[pl.ds(s * rps, rps)],
                        out_hbm.at[pl.ds(c * npad + s * rps, rps)])

    return degk(dst_all, ones, zeros)


def _prop_call(g, src_all, dst_all, zeros, npad, nchunk, d, dtype):
    rps = npad // NS
    mesh = plsc.VectorSubcoreMesh(num_cores=NC, **_MESH)

    @functools.partial(
        pl.kernel,
        mesh=mesh,
        compiler_params=_CP,
        out_type=jax.ShapeDtypeStruct((NC * npad, d), dtype),
        scratch_types=[
            pltpu.VMEM((nchunk, CH), jnp.int32),
            pltpu.VMEM((nchunk, CH), jnp.int32),
            pltpu.VMEM((CH, d), dtype),
            pltpu.VMEM((CH, d), dtype),
            pltpu.VMEM_SHARED((npad, d), dtype),
            pltpu.VMEM_SHARED((npad, d), dtype),
            pltpu.SemaphoreType.DMA,
            pltpu.SemaphoreType.DMA,
        ],
    )
    def prop(g_hbm, src_hbm, dst_hbm, z_hbm, out_hbm,
             src_v, dst_v, buf0_v, buf1_v, table_sh, acc_sh, sem0, sem1):
        c = lax.axis_index("c")
        s = lax.axis_index("s")
        wid = s * NC + c
        pltpu.sync_copy(g_hbm.at[pl.ds(s * rps, rps)],
                        table_sh.at[pl.ds(s * rps, rps)])
        pltpu.sync_copy(z_hbm.at[pl.ds(s * rps, rps)],
                        acc_sh.at[pl.ds(s * rps, rps)])
        pltpu.sync_copy(src_hbm.at[wid], src_v)
        pltpu.sync_copy(dst_hbm.at[wid], dst_v)
        plsc.subcore_barrier()

        @pl.loop(0, nchunk, step=2)
        def _(j):
            cp0 = pltpu.async_copy(table_sh.at[src_v.at[j]], buf0_v, sem0)
            cp1 = pltpu.async_copy(table_sh.at[src_v.at[j + 1]], buf1_v, sem1)
            cp0.wait()
            pltpu.sync_copy(buf0_v, acc_sh.at[dst_v.at[j]], add=True)
            cp1.wait()
            pltpu.sync_copy(buf1_v, acc_sh.at[dst_v.at[j + 1]], add=True)

        plsc.subcore_barrier()
        pltpu.sync_copy(acc_sh.at[pl.ds(s * rps, rps)],
                        out_hbm.at[pl.ds(c * npad + s * rps, rps)])

    return prop(g, src_all, dst_all, zeros)


def _tc_matmul1(x, w):
    def body(x_ref, w_ref, o_ref):
        o_ref[...] = lax.dot_general(
            x_ref[...], w_ref[...], (((1,), (0,)), ((), ())),
            precision=lax.Precision.HIGHEST,
            preferred_element_type=jnp.float32)

    return pl.pallas_call(
        body,
        out_shape=jax.ShapeDtypeStruct((x.shape[0], w.shape[1]), jnp.float32),
    )(x, w)


def _tc_scale1(degparts, h1, npad):
    n, dh = h1.shape

    def body(dp_ref, h_ref, g_ref, dinv_ref):
        deg = sum(dp_ref[i * npad:i * npad + n, 0:1] for i in range(NC)) + 1.0
        dinv = lax.rsqrt(deg)
        dinv_ref[...] = dinv
        g_ref[...] = h_ref[...] * dinv

    return pl.pallas_call(
        body,
        out_shape=[
            jax.ShapeDtypeStruct((n, dh), jnp.float32),
            jax.ShapeDtypeStruct((n, 1), jnp.float32),
        ],
    )(degparts, h1)


def _tc_layer2(s1, g1, dinv, b1, w2, npad):
    n, dh = g1.shape

    def body(s_ref, g_ref, di_ref, b_ref, w_ref, o_ref):
        ssum = sum(s_ref[i * npad:i * npad + n, 0:dh].astype(jnp.float32)
                   for i in range(NC))
        out1 = (ssum + g_ref[...]) * di_ref[...] + b_ref[...]
        h2 = lax.dot_general(
            jnp.maximum(out1, 0.0), w_ref[...], (((1,), (0,)), ((), ())),
            precision=lax.Precision.HIGHEST,
            preferred_element_type=jnp.float32)
        o_ref[...] = h2 * di_ref[...]

    return pl.pallas_call(
        body,
        out_shape=jax.ShapeDtypeStruct((n, w2.shape[1]), jnp.float32),
    )(s1, g1, dinv, b1, w2)


def _tc_final(s2, g2, dinv, b2, npad):
    n, do = g2.shape

    def body(s_ref, g_ref, di_ref, b_ref, o_ref):
        ssum = sum(s_ref[i * npad:i * npad + n, 0:do] for i in range(NC))
        o_ref[...] = (ssum + g_ref[...]) * di_ref[...] + b_ref[...]

    return pl.pallas_call(
        body,
        out_shape=jax.ShapeDtypeStruct((n, do), jnp.float32),
    )(s2, g2, dinv, b2)


def kernel(x, edge_index, W1, b1, W2, b2):
    n = x.shape[0]
    e = edge_index.shape[1]
    npad = ((n + 1 + 127) // 128) * 128
    junk = npad - n

    ew = -(-e // NW)
    ewp = -(-ew // (2 * CH)) * (2 * CH)
    pad = NW * ewp - e
    ar = jnp.arange(pad, dtype=jnp.int32)
    src_all = jnp.concatenate([edge_index[0], ar % n])
    dst_all = jnp.concatenate([edge_index[1], n + (ar % junk)])
    nchunk = ewp // CH
    src_all = src_all.reshape(NW, nchunk, CH)
    dst_all = dst_all.reshape(NW, nchunk, CH)

    ones = jnp.ones((CH, 16), jnp.float32)
    z16 = jnp.zeros((npad, 16), jnp.float32)

    degparts = _deg_call(dst_all, ones, z16, npad, nchunk)
    h1 = _tc_matmul1(x, W1)
    g1, dinv = _tc_scale1(degparts, h1, npad)
    g1p = jnp.pad(g1, ((0, npad - n), (0, 32 - g1.shape[1]))).astype(jnp.bfloat16)
    zb32 = jnp.zeros((npad, 32), jnp.bfloat16)
    s1 = _prop_call(g1p, src_all, dst_all, zb32, npad, nchunk, 32,
                    jnp.bfloat16)
    g2 = _tc_layer2(s1, g1, dinv, b1.reshape(1, -1), W2, npad)
    g2p = jnp.pad(g2, ((0, npad - n), (0, 16 - g2.shape[1])))
    s2 = _prop_call(g2p, src_all, dst_all, z16, npad, nchunk, 16,
                    jnp.float32)
    return _tc_final(s2, g2, dinv, b2.reshape(1, -1), npad)


# speedup vs baseline: 1.8517x; 1.0053x over previous
import functools

import jax
import jax.numpy as jnp
from jax import lax
from jax.experimental import pallas as pl
from jax.experimental.pallas import tpu as pltpu
from jax.experimental.pallas import tpu_sc as plsc

NC = 2
NS = 16
NW = NC * NS
CH = 2560

_MESH = dict(core_axis_name="c", subcore_axis_name="s")
_CP = pltpu.CompilerParams(use_tc_tiling_on_sc=False)


def _deg_call(dst_all, ones, zeros, npad, nchunk):
    rps = npad // NS
    mesh = plsc.VectorSubcoreMesh(num_cores=NC, **_MESH)

    @functools.partial(
        pl.kernel,
        mesh=mesh,
        compiler_params=_CP,
        out_type=jax.ShapeDtypeStruct((NC * npad, 16), jnp.float32),
        scratch_types=[
            pltpu.VMEM((nchunk, CH), jnp.int32),
            pltpu.VMEM((CH, 16), jnp.float32),
            pltpu.VMEM_SHARED((npad, 16), jnp.float32),
        ],
    )
    def degk(dst_hbm, ones_hbm, z_hbm, out_hbm, dst_v, ones_v, acc_sh):
        c = lax.axis_index("c")
        s = lax.axis_index("s")
        wid = s * NC + c
        pltpu.sync_copy(z_hbm.at[pl.ds(s * rps, rps)],
                        acc_sh.at[pl.ds(s * rps, rps)])
        pltpu.sync_copy(ones_hbm, ones_v)
        pltpu.sync_copy(dst_hbm.at[wid], dst_v)
        plsc.subcore_barrier()

        @pl.loop(0, nchunk)
        def _(j):
            pltpu.sync_copy(ones_v, acc_sh.at[dst_v.at[j]], add=True)

        plsc.subcore_barrier()
        pltpu.sync_copy(acc_sh.at[pl.ds(s * rps, rps)],
                        out_hbm.at[pl.ds(c * npad + s * rps, rps)])

    return degk(dst_all, ones, zeros)


def _prop_call(g, src_all, dst_all, zeros, npad, nchunk, d, dtype):
    rps = npad // NS
    mesh = plsc.VectorSubcoreMesh(num_cores=NC, **_MESH)

    @functools.partial(
        pl.kernel,
        mesh=mesh,
        compiler_params=_CP,
        out_type=jax.ShapeDtypeStruct((NC * npad, d), dtype),
        scratch_types=[
            pltpu.VMEM((nchunk, CH), jnp.int32),
            pltpu.VMEM((nchunk, CH), jnp.int32),
            pltpu.VMEM((CH, d), dtype),
            pltpu.VMEM((CH, d), dtype),
            pltpu.VMEM_SHARED((npad, d), dtype),
            pltpu.VMEM_SHARED((npad, d), dtype),
            pltpu.SemaphoreType.DMA,
            pltpu.SemaphoreType.DMA,
        ],
    )
    def prop(g_hbm, src_hbm, dst_hbm, z_hbm, out_hbm,
             src_v, dst_v, buf0_v, buf1_v, table_sh, acc_sh, sem0, sem1):
        c = lax.axis_index("c")
        s = lax.axis_index("s")
        wid = s * NC + c
        pltpu.sync_copy(g_hbm.at[pl.ds(s * rps, rps)],
                        table_sh.at[pl.ds(s * rps, rps)])
        pltpu.sync_copy(z_hbm.at[pl.ds(s * rps, rps)],
                        acc_sh.at[pl.ds(s * rps, rps)])
        pltpu.sync_copy(src_hbm.at[wid], src_v)
        pltpu.sync_copy(dst_hbm.at[wid], dst_v)
        plsc.subcore_barrier()

        @pl.loop(0, nchunk, step=2)
        def _(j):
            cp0 = pltpu.async_copy(table_sh.at[src_v.at[j]], buf0_v, sem0)
            cp1 = pltpu.async_copy(table_sh.at[src_v.at[j + 1]], buf1_v, sem1)
            cp0.wait()
            pltpu.sync_copy(buf0_v, acc_sh.at[dst_v.at[j]], add=True)
            cp1.wait()
            pltpu.sync_copy(buf1_v, acc_sh.at[dst_v.at[j + 1]], add=True)

        plsc.subcore_barrier()
        pltpu.sync_copy(acc_sh.at[pl.ds(s * rps, rps)],
                        out_hbm.at[pl.ds(c * npad + s * rps, rps)])

    return prop(g, src_all, dst_all, zeros)


def _tc_matmul1(x, w):
    def body(x_ref, w_ref, o_ref):
        o_ref[...] = lax.dot_general(
            x_ref[...], w_ref[...], (((1,), (0,)), ((), ())),
            precision=lax.Precision.HIGHEST,
            preferred_element_type=jnp.float32)

    return pl.pallas_call(
        body,
        out_shape=jax.ShapeDtypeStruct((x.shape[0], w.shape[1]), jnp.float32),
    )(x, w)


def _tc_scale1(degparts, h1, npad):
    n, dh = h1.shape

    def body(dp_ref, h_ref, g_ref, dinv_ref):
        deg = sum(dp_ref[i * npad:i * npad + n, 0:1] for i in range(NC)) + 1.0
        dinv = lax.rsqrt(deg)
        dinv_ref[...] = dinv
        g_ref[...] = h_ref[...] * dinv

    return pl.pallas_call(
        body,
        out_shape=[
            jax.ShapeDtypeStruct((n, dh), jnp.float32),
            jax.ShapeDtypeStruct((n, 1), jnp.float32),
        ],
    )(degparts, h1)


def _tc_layer2(s1, g1, dinv, b1, w2, npad):
    n, dh = g1.shape

    def body(s_ref, g_ref, di_ref, b_ref, w_ref, o_ref):
        ssum = sum(s_ref[i * npad:i * npad + n, 0:dh].astype(jnp.float32)
                   for i in range(NC))
        out1 = (ssum + g_ref[...]) * di_ref[...] + b_ref[...]
        h2 = lax.dot_general(
            jnp.maximum(out1, 0.0), w_ref[...], (((1,), (0,)), ((), ())),
            precision=lax.Precision.HIGHEST,
            preferred_element_type=jnp.float32)
        o_ref[...] = h2 * di_ref[...]

    return pl.pallas_call(
        body,
        out_shape=jax.ShapeDtypeStruct((n, w2.shape[1]), jnp.float32),
    )(s1, g1, dinv, b1, w2)


def _tc_final(s2, g2, dinv, b2, npad):
    n, do = g2.shape

    def body(s_ref, g_ref, di_ref, b_ref, o_ref):
        ssum = sum(s_ref[i * npad:i * npad + n, 0:do] for i in range(NC))
        o_ref[...] = (ssum + g_ref[...]) * di_ref[...] + b_ref[...]

    return pl.pallas_call(
        body,
        out_shape=jax.ShapeDtypeStruct((n, do), jnp.float32),
    )(s2, g2, dinv, b2)


def kernel(x, edge_index, W1, b1, W2, b2):
    n = x.shape[0]
    e = edge_index.shape[1]
    npad = ((n + 1 + 127) // 128) * 128
    junk = npad - n

    ew = -(-e // NW)
    ewp = -(-ew // (2 * CH)) * (2 * CH)
    pad = NW * ewp - e
    ar = jnp.arange(pad, dtype=jnp.int32)
    src_all = jnp.concatenate([edge_index[0], ar % n])
    dst_all = jnp.concatenate([edge_index[1], n + (ar % junk)])
    nchunk = ewp // CH
    src_all = src_all.reshape(NW, nchunk, CH)
    dst_all = dst_all.reshape(NW, nchunk, CH)

    ones = jnp.ones((CH, 16), jnp.float32)
    z16 = jnp.zeros((npad, 16), jnp.float32)

    degparts = _deg_call(dst_all, ones, z16, npad, nchunk)
    h1 = _tc_matmul1(x, W1)
    g1, dinv = _tc_scale1(degparts, h1, npad)
    g1p = jnp.pad(g1, ((0, npad - n), (0, 32 - g1.shape[1]))).astype(jnp.bfloat16)
    zb32 = jnp.zeros((npad, 32), jnp.bfloat16)
    s1 = _prop_call(g1p, src_all, dst_all, zb32, npad, nchunk, 32,
                    jnp.bfloat16)
    g2 = _tc_layer2(s1, g1, dinv, b1.reshape(1, -1), W2, npad)
    g2p = jnp.pad(g2, ((0, npad - n), (0, 16 - g2.shape[1])))
    s2 = _prop_call(g2p, src_all, dst_all, z16, npad, nchunk, 16,
                    jnp.float32)
    return _tc_final(s2, g2, dinv, b2.reshape(1, -1), npad)
